# baseline (speedup 1.0000x reference)
import jax
import jax.numpy as jnp
import numpy as np
from jax.experimental import pallas as pl

NUM_SPECIES = 5
EMBED = 64


def _add_kernel(a_ref, b_ref, o_ref):
    o_ref[...] = a_ref[...] + b_ref[...]


def kernel(species, x, edge_index, t, fragments_idx, W_atom, b_atom, B_rff, W1, b1, W2, b2, W_e, b_e, W_h, b_h, W_v):
    n = x.shape[0]
    onehot = jax.nn.one_hot(species, NUM_SPECIES, dtype=jnp.float32)
    h_atom = onehot @ W_atom + b_atom
    proj = 2.0 * np.pi * (t @ B_rff)
    rff = jnp.concatenate([jnp.cos(proj), jnp.sin(proj)], axis=-1)
    h_t = jax.nn.silu(rff @ W1 + b1) @ W2 + b2
    h_t = jnp.broadcast_to(h_t, (n, EMBED))
    h = jnp.concatenate([h_atom, h_t], axis=-1)
    src = edge_index[0]
    dst = edge_index[1]
    sub_graph_mask = (fragments_idx[src] == fragments_idx[dst]).astype(jnp.float32)[:, None]
    rel = x[src] - x[dst]
    d = jnp.sqrt(jnp.sum(rel * rel, axis=-1, keepdims=True) + 1e-12)
    direction = rel / d
    m_in = jnp.concatenate([h[src], h[dst], d], axis=-1)
    m = jax.nn.silu(m_in @ W_e + b_e) * sub_graph_mask
    h_agg = jax.ops.segment_sum(m, dst, num_segments=n)
    hW = h_agg @ W_h + b_h
    h_pad = jnp.pad(h, ((0, 1072), (0, 0)))
    hW_pad = jnp.pad(hW, ((0, 1072), (0, 0)))
    h_out = pl.pallas_call(
        _add_kernel,
        grid=(100,),
        in_specs=[pl.BlockSpec((512, 128), lambda i: (i, 0)),
                  pl.BlockSpec((512, 128), lambda i: (i, 0))],
        out_specs=pl.BlockSpec((512, 128), lambda i: (i, 0)),
        out_shape=jax.ShapeDtypeStruct(h_pad.shape, h.dtype),
    )(h_pad, hW_pad)[:n]
    v_feat = m @ W_v
    v_msg = (v_feat[:, :, None] * direction[:, None, :]).reshape(-1, EMBED * 3)
    v = jax.ops.segment_sum(v_msg, dst, num_segments=n).reshape(n, EMBED, 3)
    return (h_out, v)


# speedup vs baseline: 2.5239x; 2.5239x over previous
import functools

import jax
import jax.numpy as jnp
import numpy as np
from jax import lax
from jax.experimental import pallas as pl
from jax.experimental.pallas import tpu as pltpu
from jax.experimental.pallas import tpu_sc as plsc

N = 50000
E = 800000
EMBED = 64
H = 128

NC = 2
NS = 16
LANES = 16
NW = NC * NS

NPB = 1408
NPASS = 36
NPAD = NPASS * NPB
TRASH = NPB
ACC_ROWS = NPB + 16
NODE_ROWS = NPAD + 8
REC = 128

EPT = 25088
E_PAD = NW * EPT
BLK = 512
NBLK = EPT // BLK
K = 64
GR = K + 16
BUF = 27520
ZROWS = NPB // NS


def _silu(z):
    return z / (1.0 + jnp.exp(-z))


def _prep_body(t_ref, brff_ref, w1_ref, b1_ref, w2_ref, b2_ref, watom_ref,
               batom_ref, we_ref, be_ref, out_ref):
    f32 = jnp.float32
    proj = (2.0 * np.pi) * t_ref[0, 0] * brff_ref[...]
    rff = jnp.concatenate([jnp.cos(proj), jnp.sin(proj)], axis=1)
    hid = _silu(jnp.dot(rff, w1_ref[...], preferred_element_type=f32)
                + b1_ref[...])
    ht = jnp.dot(hid, w2_ref[...], preferred_element_type=f32) + b2_ref[...]
    we = we_ref[...]
    we0 = we[0:64]
    we1 = we[64:128]
    we2 = we[128:192]
    we3 = we[192:256]
    wa = watom_ref[...]
    a_tab = jnp.dot(wa, we0, preferred_element_type=f32)
    b_tab = jnp.dot(wa, we2, preferred_element_type=f32)
    c_row = (jnp.dot(batom_ref[...], we0 + we2, preferred_element_type=f32)
             + jnp.dot(ht, we1 + we3, preferred_element_type=f32)
             + be_ref[...])
    wd_row = we[256:257]
    z3 = jnp.zeros((3, 128), f32)
    ht_row = jnp.concatenate([ht, jnp.zeros((1, 64), f32)], axis=1)
    z5 = jnp.zeros((5, 128), f32)
    out_ref[...] = jnp.concatenate(
        [a_tab, z3, b_tab, z3, c_row, wd_row, ht_row, z5], axis=0)


def _prep(t, B_rff, W1, b1, W2, b2, W_atom, b_atom, W_e, b_e):
    return pl.pallas_call(
        _prep_body,
        out_shape=jax.ShapeDtypeStruct((24, 128), jnp.float32),
    )(t, B_rff, W1, b1.reshape(1, 16), W2, b2.reshape(1, 64), W_atom,
      b_atom.reshape(1, 64), W_e, b_e.reshape(1, 128))


def _rsqrt(x):
    i = plsc.bitcast(x, jnp.int32)
    i = jnp.full((LANES,), 0x5F3759DF, jnp.int32) - (i >> 1)
    y = plsc.bitcast(i, jnp.float32)
    half_x = 0.5 * x
    for _ in range(3):
        y = y * (1.5 - half_x * y * y)
    return y


def _sc_body(nodes_hbm, edges_hbm, tabs_hbm, zeros_hbm, partial_hbm,
             ebuf, blks, blkd, srcrec, dstrec, pay0, pay1, pay2, pay3,
             srcidx, dstidx, idxbuf, geof, geoi, tabs, offs,
             acc0, acc1, acc2, acc3, sem):
    core = lax.axis_index("c")
    sub = lax.axis_index("s")
    wid = core * NS + sub
    ebase = wid * EPT
    i32 = jnp.int32
    f32 = jnp.float32

    pltpu.sync_copy(tabs_hbm, tabs)

    def hist_blk(blk, cnts):
        pltpu.sync_copy(edges_hbm.at[1, pl.ds(ebase + blk * BLK, BLK)], blkd)

        def hist_vec(v, cnts):
            dstv = blkd[pl.ds(v * LANES, LANES)]
            bkt = dstv // NPB
            one = jnp.full((LANES,), 1, i32)
            zero = jnp.full((LANES,), 0, i32)
            return tuple(
                cnts[b] + jnp.where(bkt == b, one, zero)
                for b in range(NPASS))

        return lax.fori_loop(0, BLK // LANES, hist_vec, cnts)

    zero_v = jnp.zeros((LANES,), i32)
    cnt_vecs = lax.fori_loop(0, NBLK, hist_blk, (zero_v,) * NPASS)
    counts = [jnp.sum(cv) for cv in cnt_vecs]

    off = jnp.zeros((), i32)
    for b in range(NPASS):
        offs[b] = off
        seg = ((counts[b] + (K - 1)) // K) * K
        off = off + seg
    offs[NPASS] = off

    dummy = jnp.full((LANES,), TRASH << 16, i32)
    for b in range(NPASS):
        o0 = offs[b]
        nvec = (offs[b + 1] - o0) // LANES

        def prefill(v, _, o0=o0):
            ebuf[pl.ds(pl.multiple_of(o0 + v * LANES, LANES), LANES)] = dummy
            return 0

        lax.fori_loop(0, nvec, prefill, 0)

    def dist_blk(blk, ptrs):
        pltpu.sync_copy(edges_hbm.at[0, pl.ds(ebase + blk * BLK, BLK)], blks)
        pltpu.sync_copy(edges_hbm.at[1, pl.ds(ebase + blk * BLK, BLK)], blkd)

        def dist_vec(v, ptrs):
            srcv = blks[pl.ds(v * LANES, LANES)]
            dstv = blkd[pl.ds(v * LANES, LANES)]
            bkt = dstv // NPB
            new_ptrs = []
            for b in range(NPASS):
                mask = bkt == b
                cnt = jnp.sum(mask.astype(i32))
                packed = srcv | ((dstv - b * NPB) << 16)
                plsc.store_compressed(ebuf.at[pl.ds(ptrs[b], LANES)],
                                      packed, mask=mask)
                new_ptrs.append(ptrs[b] + cnt)
            return tuple(new_ptrs)

        return lax.fori_loop(0, BLK // LANES, dist_vec, ptrs)

    lax.fori_loop(0, NBLK, dist_blk, tuple(offs[b] for b in range(NPASS)))

    c_vecs = [tabs[pl.ds(2048 + j * LANES, LANES)] for j in range(H // LANES)]
    wd_vecs = [tabs[pl.ds(2176 + j * LANES, LANES)] for j in range(H // LANES)]
    lane = lax.iota(i32, LANES)
    col = [jnp.full((LANES,), c, i32) for c in range(5)]

    def run_pass(b, _):
        lo = b * NPB
        for a in (acc0, acc1, acc2, acc3):
            pltpu.sync_copy(zeros_hbm,
                            a.at[pl.ds(pl.multiple_of(sub * ZROWS, 8), ZROWS)])
        plsc.subcore_barrier()

        seg0 = offs[b]
        nch = (offs[b + 1] - seg0) // K

        def chunk(c, _):
            st = pl.multiple_of(seg0 + c * K, K)
            for j in range(K // LANES):
                w = ebuf[pl.ds(st + j * LANES, LANES)]
                sv = w & 0xFFFF
                dl = lax.shift_right_logical(w, 16)
                srcidx[pl.ds(j * LANES, LANES)] = sv
                dstidx[pl.ds(j * LANES, LANES)] = dl + lo
                idxbuf[pl.ds(j * LANES, LANES)] = dl
            pltpu.async_copy(nodes_hbm.at[srcidx], srcrec, sem).wait()
            pltpu.async_copy(nodes_hbm.at[dstidx], dstrec, sem).wait()

            for g in range(K // LANES):
                row = jnp.full((LANES,), g * LANES, i32) + lane
                xs = [plsc.load_gather(srcrec, [row, col[c]]) for c in range(3)]
                xd = [plsc.load_gather(dstrec, [row, col[c]]) for c in range(3)]
                rel = [xs[c] - xd[c] for c in range(3)]
                d2 = rel[0] * rel[0] + rel[1] * rel[1] + rel[2] * rel[2] + 1e-12
                rs = _rsqrt(d2)
                d = d2 * rs
                ss = plsc.load_gather(srcrec, [row, col[3]])
                sd = plsc.load_gather(dstrec, [row, col[3]])
                fs = plsc.load_gather(srcrec, [row, col[4]])
                fd = plsc.load_gather(dstrec, [row, col[4]])
                mk = jnp.where(fs == fd, jnp.full((LANES,), 1.0, f32),
                               jnp.zeros((LANES,), f32))
                geof[pl.ds(0 * GR + g * LANES, LANES)] = d
                geof[pl.ds(1 * GR + g * LANES, LANES)] = rel[0] * rs
                geof[pl.ds(2 * GR + g * LANES, LANES)] = rel[1] * rs
                geof[pl.ds(3 * GR + g * LANES, LANES)] = rel[2] * rs
                geof[pl.ds(4 * GR + g * LANES, LANES)] = mk
                geoi[pl.ds(0 * GR + g * LANES, LANES)] = ss.astype(i32)
                geoi[pl.ds(1 * GR + g * LANES, LANES)] = sd.astype(i32)

            def edge(e, _):
                ev = jnp.full((LANES,), e, i32)
                de = plsc.load_gather(geof, [ev])
                dxe = plsc.load_gather(geof, [ev + GR])
                dye = plsc.load_gather(geof, [ev + 2 * GR])
                dze = plsc.load_gather(geof, [ev + 3 * GR])
                mke = plsc.load_gather(geof, [ev + 4 * GR])
                ssv = plsc.load_gather(geoi, [ev])
                sdv = plsc.load_gather(geoi, [ev + GR])
                a_idx = ssv * H + lane
                b_idx = sdv * H + (1024 + lane)
                for j in range(H // LANES):
                    z = (plsc.load_gather(tabs, [a_idx + j * LANES])
                         + plsc.load_gather(tabs, [b_idx + j * LANES])
                         + c_vecs[j] + de * wd_vecs[j])
                    m = (z / (1.0 + jnp.exp(-z))) * mke
                    pay0[e, pl.ds(j * LANES, LANES)] = m
                    pay1[e, pl.ds(j * LANES, LANES)] = m * dxe
                    pay2[e, pl.ds(j * LANES, LANES)] = m * dye
                    pay3[e, pl.ds(j * LANES, LANES)] = m * dze
                return 0

            lax.fori_loop(0, K, edge, 0)

            pltpu.sync_copy(pay0, acc0.at[idxbuf], add=True)
            pltpu.sync_copy(pay1, acc1.at[idxbuf], add=True)
            pltpu.sync_copy(pay2, acc2.at[idxbuf], add=True)
            pltpu.sync_copy(pay3, acc3.at[idxbuf], add=True)
            return 0

        lax.fori_loop(0, nch, chunk, 0)
        plsc.subcore_barrier()

        for ci, a in enumerate((acc0, acc1, acc2, acc3)):
            pltpu.sync_copy(
                a.at[pl.ds(pl.multiple_of(sub * (NPB // NS), 8), NPB // NS)],
                partial_hbm.at[core, ci, pl.ds(pl.multiple_of(
                    b * NPB + sub * (NPB // NS), 8), NPB // NS)])
        plsc.subcore_barrier()
        return 0

    lax.fori_loop(0, NPASS, run_pass, 0)


@functools.partial(jax.jit, static_argnums=())
def _sc_edges(nodes, edges, tabs_flat, zrows):
    mesh = plsc.VectorSubcoreMesh(core_axis_name="c", subcore_axis_name="s")
    return pl.kernel(
        _sc_body,
        out_type=jax.ShapeDtypeStruct((NC, 4, NPAD, H), jnp.float32),
        mesh=mesh,
        compiler_params=pltpu.CompilerParams(needs_layout_passes=False),
        scratch_types=[
            pltpu.VMEM((BUF,), jnp.int32),
            pltpu.VMEM((BLK,), jnp.int32),
            pltpu.VMEM((BLK,), jnp.int32),
            pltpu.VMEM((K, REC), jnp.float32),
            pltpu.VMEM((K, REC), jnp.float32),
            pltpu.VMEM((K, H), jnp.float32),
            pltpu.VMEM((K, H), jnp.float32),
            pltpu.VMEM((K, H), jnp.float32),
            pltpu.VMEM((K, H), jnp.float32),
            pltpu.VMEM((K,), jnp.int32),
            pltpu.VMEM((K,), jnp.int32),
            pltpu.VMEM((K,), jnp.int32),
            pltpu.VMEM((5 * GR,), jnp.float32),
            pltpu.VMEM((2 * GR,), jnp.int32),
            pltpu.VMEM((3072,), jnp.float32),
            pltpu.SMEM((40,), jnp.int32),
            pltpu.VMEM_SHARED((ACC_ROWS, H), jnp.float32),
            pltpu.VMEM_SHARED((ACC_ROWS, H), jnp.float32),
            pltpu.VMEM_SHARED((ACC_ROWS, H), jnp.float32),
            pltpu.VMEM_SHARED((ACC_ROWS, H), jnp.float32),
            pltpu.SemaphoreType.DMA,
        ],
    )(nodes, edges, tabs_flat, zrows)


def _post_body(partial_ref, species_ref, tabs_ref, watom_ref, wh_ref, bh_ref,
               wv3_ref, hout_ref, vflat_ref):
    f32 = jnp.float32
    p = partial_ref[...]
    g = p[0] + p[1]
    gm = g[0]
    gcat = jnp.concatenate([g[1], g[2], g[3]], axis=1)
    s = species_ref[0, 0, :]
    onehot = (s[:, None] == lax.broadcasted_iota(jnp.int32, (s.shape[0], 8), 1)
              ).astype(f32)
    h_atom = jnp.dot(onehot, watom_ref[...], preferred_element_type=f32)
    ht = tabs_ref[18:19, 0:EMBED]
    h = jnp.concatenate(
        [h_atom, jnp.broadcast_to(ht, (s.shape[0], EMBED))], axis=1)
    hout_ref[...] = (h + jnp.dot(gm, wh_ref[...],
                                 preferred_element_type=f32) + bh_ref[...])
    vflat_ref[...] = jnp.dot(gcat, wv3_ref[...], preferred_element_type=f32)


def _post(partial, species_r, tabs24, watom_pad, W_h, b_h, wv3):
    bn = 512
    nblk = NPAD // bn
    return pl.pallas_call(
        _post_body,
        grid=(nblk,),
        in_specs=[
            pl.BlockSpec((NC, 4, bn, H), lambda i: (0, 0, i, 0)),
            pl.BlockSpec((1, 1, bn), lambda i: (i, 0, 0)),
            pl.BlockSpec((24, 128), lambda i: (0, 0)),
            pl.BlockSpec((8, EMBED), lambda i: (0, 0)),
            pl.BlockSpec((H, H), lambda i: (0, 0)),
            pl.BlockSpec((1, H), lambda i: (0, 0)),
            pl.BlockSpec((3 * H, 3 * EMBED), lambda i: (0, 0)),
        ],
        out_specs=[
            pl.BlockSpec((bn, H), lambda i: (i, 0)),
            pl.BlockSpec((bn, 3 * EMBED), lambda i: (i, 0)),
        ],
        out_shape=[
            jax.ShapeDtypeStruct((NPAD, H), jnp.float32),
            jax.ShapeDtypeStruct((NPAD, 3 * EMBED), jnp.float32),
        ],
    )(partial, species_r, tabs24, watom_pad, W_h, b_h, wv3)


def kernel(species, x, edge_index, t, fragments_idx, W_atom, b_atom, B_rff,
           W1, b1, W2, b2, W_e, b_e, W_h, b_h, W_v):
    f32 = jnp.float32
    i32 = jnp.int32

    tabs24 = _prep(t, B_rff, W1, b1, W2, b2, W_atom, b_atom, W_e, b_e)
    tabs_flat = tabs24.reshape(3072)

    nodes = jnp.zeros((NODE_ROWS, REC), f32)
    nodes = nodes.at[:N, 0:3].set(x)
    nodes = nodes.at[:N, 3].set(species.astype(f32))
    nodes = nodes.at[:N, 4].set(fragments_idx.astype(f32))

    ei = edge_index.astype(i32)
    pad = jnp.concatenate(
        [jnp.zeros((1, E_PAD - E), i32),
         jnp.full((1, E_PAD - E), NPAD, i32)], axis=0)
    edges = jnp.concatenate([ei, pad], axis=1)

    zrows = jnp.zeros((ZROWS, H), f32)
    partial = _sc_edges(nodes, edges, tabs_flat, zrows)

    wv3 = jnp.zeros((3 * H, 3 * EMBED), f32)
    for c in range(3):
        wv3 = wv3.at[c * H:(c + 1) * H, c::3].set(W_v)

    species_r = jnp.zeros((NPAD,), i32).at[:N].set(species.astype(i32))
    species_r = species_r.reshape(NPAD // 512, 1, 512)
    watom_pad = jnp.concatenate([W_atom, jnp.zeros((3, EMBED), f32)], axis=0)

    h_out, v_flat = _post(partial, species_r, tabs24, watom_pad, W_h,
                          b_h.reshape(1, H), wv3)
    return (h_out[:N], v_flat[:N].reshape(N, EMBED, 3))


# speedup vs baseline: 2.6172x; 1.0369x over previous
import functools

import jax
import jax.numpy as jnp
import numpy as np
from jax import lax
from jax.experimental import pallas as pl
from jax.experimental.pallas import tpu as pltpu
from jax.experimental.pallas import tpu_sc as plsc

N = 50000
E = 800000
EMBED = 64
H = 128

NC = 2
NS = 16
LANES = 16
NW = NC * NS

NPB = 1408
NPASS = 36
NPAD = NPASS * NPB
TRASH = NPB
ACC_ROWS = NPB + 16
NODE_ROWS = NPAD + 8
REC = 128

EPT = 25088
E_PAD = NW * EPT
BLK = 512
NBLK = EPT // BLK
K = 64
GR = K + 16
BUF = 27520
ZROWS = NPB // NS


def _silu(z):
    return z / (1.0 + jnp.exp(-z))


def _prep_body(t_ref, brff_ref, w1_ref, b1_ref, w2_ref, b2_ref, watom_ref,
               batom_ref, we_ref, be_ref, out_ref):
    f32 = jnp.float32
    proj = (2.0 * np.pi) * t_ref[0, 0] * brff_ref[...]
    rff = jnp.concatenate([jnp.cos(proj), jnp.sin(proj)], axis=1)
    hid = _silu(jnp.dot(rff, w1_ref[...], preferred_element_type=f32)
                + b1_ref[...])
    ht = jnp.dot(hid, w2_ref[...], preferred_element_type=f32) + b2_ref[...]
    we = we_ref[...]
    we0 = we[0:64]
    we1 = we[64:128]
    we2 = we[128:192]
    we3 = we[192:256]
    wa = watom_ref[...]
    a_tab = jnp.dot(wa, we0, preferred_element_type=f32)
    b_tab = jnp.dot(wa, we2, preferred_element_type=f32)
    c_row = (jnp.dot(batom_ref[...], we0 + we2, preferred_element_type=f32)
             + jnp.dot(ht, we1 + we3, preferred_element_type=f32)
             + be_ref[...])
    wd_row = we[256:257]
    z3 = jnp.zeros((3, 128), f32)
    ht_row = jnp.concatenate([ht, jnp.zeros((1, 64), f32)], axis=1)
    z5 = jnp.zeros((5, 128), f32)
    out_ref[...] = jnp.concatenate(
        [a_tab, z3, b_tab, z3, c_row, wd_row, ht_row, z5], axis=0)


def _prep(t, B_rff, W1, b1, W2, b2, W_atom, b_atom, W_e, b_e):
    return pl.pallas_call(
        _prep_body,
        out_shape=jax.ShapeDtypeStruct((24, 128), jnp.float32),
    )(t, B_rff, W1, b1.reshape(1, 16), W2, b2.reshape(1, 64), W_atom,
      b_atom.reshape(1, 64), W_e, b_e.reshape(1, 128))


def _rsqrt(x):
    i = plsc.bitcast(x, jnp.int32)
    i = jnp.full((LANES,), 0x5F3759DF, jnp.int32) - (i >> 1)
    y = plsc.bitcast(i, jnp.float32)
    half_x = 0.5 * x
    for _ in range(3):
        y = y * (1.5 - half_x * y * y)
    return y


def _sc_body(nodes_hbm, edges_hbm, tabs_hbm, zeros_hbm, partial_hbm,
             ebuf, blks, blkd, srcrec, dstrec, pay0, pay1, pay2, pay3,
             srcidx, dstidx, idxbuf, geof, geoi, tabs, offs,
             acc0, acc1, acc2, acc3, sem, sem2):
    core = lax.axis_index("c")
    sub = lax.axis_index("s")
    wid = core * NS + sub
    ebase = wid * EPT
    i32 = jnp.int32
    f32 = jnp.float32

    pltpu.sync_copy(tabs_hbm, tabs)

    def hist_blk(blk, cnts):
        pltpu.sync_copy(edges_hbm.at[1, pl.ds(ebase + blk * BLK, BLK)], blkd)

        def hist_vec(v, cnts):
            dstv = blkd[pl.ds(v * LANES, LANES)]
            bkt = dstv // NPB
            one = jnp.full((LANES,), 1, i32)
            zero = jnp.full((LANES,), 0, i32)
            return tuple(
                cnts[b] + jnp.where(bkt == b, one, zero)
                for b in range(NPASS))

        return lax.fori_loop(0, BLK // LANES, hist_vec, cnts)

    zero_v = jnp.zeros((LANES,), i32)
    cnt_vecs = lax.fori_loop(0, NBLK, hist_blk, (zero_v,) * NPASS)
    counts = [jnp.sum(cv) for cv in cnt_vecs]

    off = jnp.zeros((), i32)
    for b in range(NPASS):
        offs[b] = off
        seg = ((counts[b] + (K - 1)) // K) * K
        off = off + seg
    offs[NPASS] = off

    dummy = jnp.full((LANES,), TRASH << 16, i32)
    for b in range(NPASS):
        o0 = offs[b]
        nvec = (offs[b + 1] - o0) // LANES

        def prefill(v, _, o0=o0):
            ebuf[pl.ds(pl.multiple_of(o0 + v * LANES, LANES), LANES)] = dummy
            return 0

        lax.fori_loop(0, nvec, prefill, 0)

    def dist_blk(blk, ptrs):
        pltpu.sync_copy(edges_hbm.at[0, pl.ds(ebase + blk * BLK, BLK)], blks)
        pltpu.sync_copy(edges_hbm.at[1, pl.ds(ebase + blk * BLK, BLK)], blkd)

        def dist_vec(v, ptrs):
            srcv = blks[pl.ds(v * LANES, LANES)]
            dstv = blkd[pl.ds(v * LANES, LANES)]
            bkt = dstv // NPB
            new_ptrs = []
            for b in range(NPASS):
                mask = bkt == b
                cnt = jnp.sum(mask.astype(i32))
                packed = srcv | ((dstv - b * NPB) << 16)
                plsc.store_compressed(ebuf.at[pl.ds(ptrs[b], LANES)],
                                      packed, mask=mask)
                new_ptrs.append(ptrs[b] + cnt)
            return tuple(new_ptrs)

        return lax.fori_loop(0, BLK // LANES, dist_vec, ptrs)

    lax.fori_loop(0, NBLK, dist_blk, tuple(offs[b] for b in range(NPASS)))

    c_vecs = [tabs[pl.ds(2048 + j * LANES, LANES)] for j in range(H // LANES)]
    wd_vecs = [tabs[pl.ds(2176 + j * LANES, LANES)] for j in range(H // LANES)]
    lane = lax.iota(i32, LANES)
    col = [jnp.full((LANES,), c, i32) for c in range(5)]

    def run_pass(b, _):
        lo = b * NPB
        for a in (acc0, acc1, acc2, acc3):
            pltpu.sync_copy(zeros_hbm,
                            a.at[pl.ds(pl.multiple_of(sub * ZROWS, 8), ZROWS)])
        plsc.subcore_barrier()

        seg0 = offs[b]
        nch = (offs[b + 1] - seg0) // K

        def chunk(c, _):
            st = pl.multiple_of(seg0 + c * K, K)
            for j in range(K // LANES):
                w = ebuf[pl.ds(st + j * LANES, LANES)]
                sv = w & 0xFFFF
                dl = lax.shift_right_logical(w, 16)
                srcidx[pl.ds(j * LANES, LANES)] = sv
                dstidx[pl.ds(j * LANES, LANES)] = dl + lo
                idxbuf[pl.ds(j * LANES, LANES)] = dl
            ca = pltpu.async_copy(nodes_hbm.at[srcidx], srcrec, sem)
            cb = pltpu.async_copy(nodes_hbm.at[dstidx], dstrec, sem2)
            ca.wait()
            cb.wait()

            for g in range(K // LANES):
                row = jnp.full((LANES,), g * LANES, i32) + lane
                xs = [plsc.load_gather(srcrec, [row, col[c]]) for c in range(3)]
                xd = [plsc.load_gather(dstrec, [row, col[c]]) for c in range(3)]
                rel = [xs[c] - xd[c] for c in range(3)]
                d2 = rel[0] * rel[0] + rel[1] * rel[1] + rel[2] * rel[2] + 1e-12
                rs = _rsqrt(d2)
                d = d2 * rs
                ss = plsc.load_gather(srcrec, [row, col[3]])
                sd = plsc.load_gather(dstrec, [row, col[3]])
                fs = plsc.load_gather(srcrec, [row, col[4]])
                fd = plsc.load_gather(dstrec, [row, col[4]])
                mk = jnp.where(fs == fd, jnp.full((LANES,), 1.0, f32),
                               jnp.zeros((LANES,), f32))
                geof[pl.ds(0 * GR + g * LANES, LANES)] = d
                geof[pl.ds(1 * GR + g * LANES, LANES)] = rel[0] * rs
                geof[pl.ds(2 * GR + g * LANES, LANES)] = rel[1] * rs
                geof[pl.ds(3 * GR + g * LANES, LANES)] = rel[2] * rs
                geof[pl.ds(4 * GR + g * LANES, LANES)] = mk
                geoi[pl.ds(0 * GR + g * LANES, LANES)] = ss.astype(i32)
                geoi[pl.ds(1 * GR + g * LANES, LANES)] = sd.astype(i32)

            def edge(e, _):
                ev = jnp.full((LANES,), e, i32)
                de = plsc.load_gather(geof, [ev])
                dxe = plsc.load_gather(geof, [ev + GR])
                dye = plsc.load_gather(geof, [ev + 2 * GR])
                dze = plsc.load_gather(geof, [ev + 3 * GR])
                mke = plsc.load_gather(geof, [ev + 4 * GR])
                ssv = plsc.load_gather(geoi, [ev])
                sdv = plsc.load_gather(geoi, [ev + GR])
                a_idx = ssv * H + lane
                b_idx = sdv * H + (1024 + lane)
                for j in range(H // LANES):
                    z = (plsc.load_gather(tabs, [a_idx + j * LANES])
                         + plsc.load_gather(tabs, [b_idx + j * LANES])
                         + c_vecs[j] + de * wd_vecs[j])
                    m = (z / (1.0 + jnp.exp(-z))) * mke
                    pay0[e, pl.ds(j * LANES, LANES)] = m
                    pay1[e, pl.ds(j * LANES, LANES)] = m * dxe
                    pay2[e, pl.ds(j * LANES, LANES)] = m * dye
                    pay3[e, pl.ds(j * LANES, LANES)] = m * dze
                return 0

            lax.fori_loop(0, K, edge, 0)

            pltpu.sync_copy(pay0, acc0.at[idxbuf], add=True)
            pltpu.sync_copy(pay1, acc1.at[idxbuf], add=True)
            pltpu.sync_copy(pay2, acc2.at[idxbuf], add=True)
            pltpu.sync_copy(pay3, acc3.at[idxbuf], add=True)
            return 0

        lax.fori_loop(0, nch, chunk, 0)
        plsc.subcore_barrier()

        for ci, a in enumerate((acc0, acc1, acc2, acc3)):
            pltpu.sync_copy(
                a.at[pl.ds(pl.multiple_of(sub * (NPB // NS), 8), NPB // NS)],
                partial_hbm.at[core, ci, pl.ds(pl.multiple_of(
                    b * NPB + sub * (NPB // NS), 8), NPB // NS)])
        plsc.subcore_barrier()
        return 0

    lax.fori_loop(0, NPASS, run_pass, 0)


@functools.partial(jax.jit, static_argnums=())
def _sc_edges(nodes, edges, tabs_flat, zrows):
    mesh = plsc.VectorSubcoreMesh(core_axis_name="c", subcore_axis_name="s")
    return pl.kernel(
        _sc_body,
        out_type=jax.ShapeDtypeStruct((NC, 4, NPAD, H), jnp.float32),
        mesh=mesh,
        compiler_params=pltpu.CompilerParams(needs_layout_passes=False),
        scratch_types=[
            pltpu.VMEM((BUF,), jnp.int32),
            pltpu.VMEM((BLK,), jnp.int32),
            pltpu.VMEM((BLK,), jnp.int32),
            pltpu.VMEM((K, REC), jnp.float32),
            pltpu.VMEM((K, REC), jnp.float32),
            pltpu.VMEM((K, H), jnp.float32),
            pltpu.VMEM((K, H), jnp.float32),
            pltpu.VMEM((K, H), jnp.float32),
            pltpu.VMEM((K, H), jnp.float32),
            pltpu.VMEM((K,), jnp.int32),
            pltpu.VMEM((K,), jnp.int32),
            pltpu.VMEM((K,), jnp.int32),
            pltpu.VMEM((5 * GR,), jnp.float32),
            pltpu.VMEM((2 * GR,), jnp.int32),
            pltpu.VMEM((3072,), jnp.float32),
            pltpu.SMEM((40,), jnp.int32),
            pltpu.VMEM_SHARED((ACC_ROWS, H), jnp.float32),
            pltpu.VMEM_SHARED((ACC_ROWS, H), jnp.float32),
            pltpu.VMEM_SHARED((ACC_ROWS, H), jnp.float32),
            pltpu.VMEM_SHARED((ACC_ROWS, H), jnp.float32),
            pltpu.SemaphoreType.DMA,
            pltpu.SemaphoreType.DMA,
        ],
    )(nodes, edges, tabs_flat, zrows)


def _post_body(partial_ref, species_ref, tabs_ref, watom_ref, wh_ref, bh_ref,
               wv3_ref, hout_ref, vflat_ref):
    f32 = jnp.float32
    p = partial_ref[...]
    g = p[0] + p[1]
    gm = g[0]
    gcat = jnp.concatenate([g[1], g[2], g[3]], axis=1)
    s = species_ref[0, 0, :]
    onehot = (s[:, None] == lax.broadcasted_iota(jnp.int32, (s.shape[0], 8), 1)
              ).astype(f32)
    h_atom = jnp.dot(onehot, watom_ref[...], preferred_element_type=f32)
    ht = tabs_ref[18:19, 0:EMBED]
    h = jnp.concatenate(
        [h_atom, jnp.broadcast_to(ht, (s.shape[0], EMBED))], axis=1)
    hout_ref[...] = (h + jnp.dot(gm, wh_ref[...],
                                 preferred_element_type=f32) + bh_ref[...])
    vflat_ref[...] = jnp.dot(gcat, wv3_ref[...], preferred_element_type=f32)


def _post(partial, species_r, tabs24, watom_pad, W_h, b_h, wv3):
    bn = 512
    nblk = NPAD // bn
    return pl.pallas_call(
        _post_body,
        grid=(nblk,),
        in_specs=[
            pl.BlockSpec((NC, 4, bn, H), lambda i: (0, 0, i, 0)),
            pl.BlockSpec((1, 1, bn), lambda i: (i, 0, 0)),
            pl.BlockSpec((24, 128), lambda i: (0, 0)),
            pl.BlockSpec((8, EMBED), lambda i: (0, 0)),
            pl.BlockSpec((H, H), lambda i: (0, 0)),
            pl.BlockSpec((1, H), lambda i: (0, 0)),
            pl.BlockSpec((3 * H, 3 * EMBED), lambda i: (0, 0)),
        ],
        out_specs=[
            pl.BlockSpec((bn, H), lambda i: (i, 0)),
            pl.BlockSpec((bn, 3 * EMBED), lambda i: (i, 0)),
        ],
        out_shape=[
            jax.ShapeDtypeStruct((NPAD, H), jnp.float32),
            jax.ShapeDtypeStruct((NPAD, 3 * EMBED), jnp.float32),
        ],
    )(partial, species_r, tabs24, watom_pad, W_h, b_h, wv3)


def kernel(species, x, edge_index, t, fragments_idx, W_atom, b_atom, B_rff,
           W1, b1, W2, b2, W_e, b_e, W_h, b_h, W_v):
    f32 = jnp.float32
    i32 = jnp.int32

    tabs24 = _prep(t, B_rff, W1, b1, W2, b2, W_atom, b_atom, W_e, b_e)
    tabs_flat = tabs24.reshape(3072)

    nodes = jnp.zeros((NODE_ROWS, REC), f32)
    nodes = nodes.at[:N, 0:3].set(x)
    nodes = nodes.at[:N, 3].set(species.astype(f32))
    nodes = nodes.at[:N, 4].set(fragments_idx.astype(f32))

    ei = edge_index.astype(i32)
    pad = jnp.concatenate(
        [jnp.zeros((1, E_PAD - E), i32),
         jnp.full((1, E_PAD - E), NPAD, i32)], axis=0)
    edges = jnp.concatenate([ei, pad], axis=1)

    zrows = jnp.zeros((ZROWS, H), f32)
    partial = _sc_edges(nodes, edges, tabs_flat, zrows)

    wv3 = jnp.zeros((3 * H, 3 * EMBED), f32)
    for c in range(3):
        wv3 = wv3.at[c * H:(c + 1) * H, c::3].set(W_v)

    species_r = jnp.zeros((NPAD,), i32).at[:N].set(species.astype(i32))
    species_r = species_r.reshape(NPAD // 512, 1, 512)
    watom_pad = jnp.concatenate([W_atom, jnp.zeros((3, EMBED), f32)], axis=0)

    h_out, v_flat = _post(partial, species_r, tabs24, watom_pad, W_h,
                          b_h.reshape(1, H), wv3)
    return (h_out[:N], v_flat[:N].reshape(N, EMBED, 3))


# speedup vs baseline: 2.8194x; 1.0773x over previous
import functools

import jax
import jax.numpy as jnp
import numpy as np
from jax import lax
from jax.experimental import pallas as pl
from jax.experimental.pallas import tpu as pltpu
from jax.experimental.pallas import tpu_sc as plsc

N = 50000
E = 800000
EMBED = 64
H = 128

NC = 2
NS = 16
LANES = 16
NW = NC * NS

NPB = 1408
NPASS = 36
NPAD = NPASS * NPB
TRASH = NPB
ACC_ROWS = NPB + 16
NODE_ROWS = NPAD + 8
REC = 128

EPT = 25088
E_PAD = NW * EPT
BLK = 512
NBLK = EPT // BLK
K = 64
GR = K + 16
BUF = 27520
ZROWS = NPB // NS


def _silu(z):
    return z / (1.0 + jnp.exp(-z))


def _prep_body(t_ref, brff_ref, w1_ref, b1_ref, w2_ref, b2_ref, watom_ref,
               batom_ref, we_ref, be_ref, out_ref):
    f32 = jnp.float32
    proj = (2.0 * np.pi) * t_ref[0, 0] * brff_ref[...]
    rff = jnp.concatenate([jnp.cos(proj), jnp.sin(proj)], axis=1)
    hid = _silu(jnp.dot(rff, w1_ref[...], preferred_element_type=f32)
                + b1_ref[...])
    ht = jnp.dot(hid, w2_ref[...], preferred_element_type=f32) + b2_ref[...]
    we = we_ref[...]
    we0 = we[0:64]
    we1 = we[64:128]
    we2 = we[128:192]
    we3 = we[192:256]
    wa = watom_ref[...]
    a_tab = jnp.dot(wa, we0, preferred_element_type=f32)
    b_tab = jnp.dot(wa, we2, preferred_element_type=f32)
    c_row = (jnp.dot(batom_ref[...], we0 + we2, preferred_element_type=f32)
             + jnp.dot(ht, we1 + we3, preferred_element_type=f32)
             + be_ref[...])
    wd_row = we[256:257]
    ht_row = jnp.concatenate([ht, jnp.zeros((1, 64), f32)], axis=1)
    ab = (jnp.repeat(a_tab, 5, axis=0) + jnp.tile(b_tab, (5, 1))
          + jnp.broadcast_to(c_row, (25, 128)))
    z5 = jnp.zeros((5, 128), f32)
    out_ref[...] = jnp.concatenate([ab, wd_row, ht_row, z5], axis=0)


def _prep(t, B_rff, W1, b1, W2, b2, W_atom, b_atom, W_e, b_e):
    return pl.pallas_call(
        _prep_body,
        out_shape=jax.ShapeDtypeStruct((32, 128), jnp.float32),
    )(t, B_rff, W1, b1.reshape(1, 16), W2, b2.reshape(1, 64), W_atom,
      b_atom.reshape(1, 64), W_e, b_e.reshape(1, 128))


def _rsqrt(x):
    i = plsc.bitcast(x, jnp.int32)
    i = jnp.full((LANES,), 0x5F3759DF, jnp.int32) - (i >> 1)
    y = plsc.bitcast(i, jnp.float32)
    half_x = 0.5 * x
    for _ in range(3):
        y = y * (1.5 - half_x * y * y)
    return y


def _sc_body(nodes_hbm, edges_hbm, tabs_hbm, zeros_hbm, partial_hbm,
             ebuf, blks, blkd, srcrec, dstrec, pay0, pay1, pay2, pay3,
             srcidx, dstidx, idxbuf, geof, geoi, tabs, offs,
             acc0, acc1, acc2, acc3, sem, sem2):
    core = lax.axis_index("c")
    sub = lax.axis_index("s")
    wid = core * NS + sub
    ebase = wid * EPT
    i32 = jnp.int32
    f32 = jnp.float32

    pltpu.sync_copy(tabs_hbm, tabs)

    def hist_blk(blk, cnts):
        pltpu.sync_copy(edges_hbm.at[1, pl.ds(ebase + blk * BLK, BLK)], blkd)

        def hist_vec(v, cnts):
            dstv = blkd[pl.ds(v * LANES, LANES)]
            bkt = dstv // NPB
            one = jnp.full((LANES,), 1, i32)
            zero = jnp.full((LANES,), 0, i32)
            return tuple(
                cnts[b] + jnp.where(bkt == b, one, zero)
                for b in range(NPASS))

        return lax.fori_loop(0, BLK // LANES, hist_vec, cnts)

    zero_v = jnp.zeros((LANES,), i32)
    cnt_vecs = lax.fori_loop(0, NBLK, hist_blk, (zero_v,) * NPASS)
    counts = [jnp.sum(cv) for cv in cnt_vecs]

    off = jnp.zeros((), i32)
    for b in range(NPASS):
        offs[b] = off
        seg = ((counts[b] + (K - 1)) // K) * K
        off = off + seg
    offs[NPASS] = off

    dummy = jnp.full((LANES,), TRASH << 16, i32)
    for b in range(NPASS):
        o0 = offs[b]
        nvec = (offs[b + 1] - o0) // LANES

        def prefill(v, _, o0=o0):
            ebuf[pl.ds(pl.multiple_of(o0 + v * LANES, LANES), LANES)] = dummy
            return 0

        lax.fori_loop(0, nvec, prefill, 0)

    def dist_blk(blk, ptrs):
        pltpu.sync_copy(edges_hbm.at[0, pl.ds(ebase + blk * BLK, BLK)], blks)
        pltpu.sync_copy(edges_hbm.at[1, pl.ds(ebase + blk * BLK, BLK)], blkd)

        def dist_vec(v, ptrs):
            srcv = blks[pl.ds(v * LANES, LANES)]
            dstv = blkd[pl.ds(v * LANES, LANES)]
            bkt = dstv // NPB
            new_ptrs = []
            for b in range(NPASS):
                mask = bkt == b
                cnt = jnp.sum(mask.astype(i32))
                packed = srcv | ((dstv - b * NPB) << 16)
                plsc.store_compressed(ebuf.at[pl.ds(ptrs[b], LANES)],
                                      packed, mask=mask)
                new_ptrs.append(ptrs[b] + cnt)
            return tuple(new_ptrs)

        return lax.fori_loop(0, BLK // LANES, dist_vec, ptrs)

    lax.fori_loop(0, NBLK, dist_blk, tuple(offs[b] for b in range(NPASS)))

    wd_vecs = [tabs[pl.ds(25 * H + j * LANES, LANES)] for j in range(H // LANES)]
    lane = lax.iota(i32, LANES)
    col = [jnp.full((LANES,), c, i32) for c in range(5)]

    def run_pass(b, _):
        lo = b * NPB
        for a in (acc0, acc1, acc2, acc3):
            pltpu.sync_copy(zeros_hbm,
                            a.at[pl.ds(pl.multiple_of(sub * ZROWS, 8), ZROWS)])
        plsc.subcore_barrier()

        seg0 = offs[b]
        nch = (offs[b + 1] - seg0) // K

        def chunk(c, _):
            st = pl.multiple_of(seg0 + c * K, K)
            for j in range(K // LANES):
                w = ebuf[pl.ds(st + j * LANES, LANES)]
                sv = w & 0xFFFF
                dl = lax.shift_right_logical(w, 16)
                srcidx[pl.ds(j * LANES, LANES)] = sv
                dstidx[pl.ds(j * LANES, LANES)] = dl + lo
                idxbuf[pl.ds(j * LANES, LANES)] = dl
            ca = pltpu.async_copy(nodes_hbm.at[srcidx], srcrec, sem)
            cb = pltpu.async_copy(nodes_hbm.at[dstidx], dstrec, sem2)
            ca.wait()
            cb.wait()

            for g in range(K // LANES):
                row = jnp.full((LANES,), g * LANES, i32) + lane
                xs = [plsc.load_gather(srcrec, [row, col[c]]) for c in range(3)]
                xd = [plsc.load_gather(dstrec, [row, col[c]]) for c in range(3)]
                rel = [xs[c] - xd[c] for c in range(3)]
                d2 = rel[0] * rel[0] + rel[1] * rel[1] + rel[2] * rel[2] + 1e-12
                rs = _rsqrt(d2)
                d = d2 * rs
                ss = plsc.load_gather(srcrec, [row, col[3]])
                sd = plsc.load_gather(dstrec, [row, col[3]])
                fs = plsc.load_gather(srcrec, [row, col[4]])
                fd = plsc.load_gather(dstrec, [row, col[4]])
                mk = jnp.where(fs == fd, jnp.full((LANES,), 1.0, f32),
                               jnp.zeros((LANES,), f32))
                geof[pl.ds(0 * GR + g * LANES, LANES)] = d
                geof[pl.ds(1 * GR + g * LANES, LANES)] = rel[0] * rs
                geof[pl.ds(2 * GR + g * LANES, LANES)] = rel[1] * rs
                geof[pl.ds(3 * GR + g * LANES, LANES)] = rel[2] * rs
                geof[pl.ds(4 * GR + g * LANES, LANES)] = mk
                geoi[pl.ds(0 * GR + g * LANES, LANES)] = (
                    ss.astype(i32) * 5 + sd.astype(i32))

            def edge(e, _):
                ev = jnp.full((LANES,), e, i32)
                de = plsc.load_gather(geof, [ev])
                dxe = plsc.load_gather(geof, [ev + GR])
                dye = plsc.load_gather(geof, [ev + 2 * GR])
                dze = plsc.load_gather(geof, [ev + 3 * GR])
                mke = plsc.load_gather(geof, [ev + 4 * GR])
                combv = plsc.load_gather(geoi, [ev])
                ab_idx = combv * H + lane
                for j in range(H // LANES):
                    z = (plsc.load_gather(tabs, [ab_idx + j * LANES])
                         + de * wd_vecs[j])
                    m = (z / (1.0 + jnp.exp(-z))) * mke
                    pay0[e, pl.ds(j * LANES, LANES)] = m
                    pay1[e, pl.ds(j * LANES, LANES)] = m * dxe
                    pay2[e, pl.ds(j * LANES, LANES)] = m * dye
                    pay3[e, pl.ds(j * LANES, LANES)] = m * dze
                return 0

            lax.fori_loop(0, K, edge, 0)

            pltpu.sync_copy(pay0, acc0.at[idxbuf], add=True)
            pltpu.sync_copy(pay1, acc1.at[idxbuf], add=True)
            pltpu.sync_copy(pay2, acc2.at[idxbuf], add=True)
            pltpu.sync_copy(pay3, acc3.at[idxbuf], add=True)
            return 0

        lax.fori_loop(0, nch, chunk, 0)
        plsc.subcore_barrier()

        for ci, a in enumerate((acc0, acc1, acc2, acc3)):
            pltpu.sync_copy(
                a.at[pl.ds(pl.multiple_of(sub * (NPB // NS), 8), NPB // NS)],
                partial_hbm.at[core, ci, pl.ds(pl.multiple_of(
                    b * NPB + sub * (NPB // NS), 8), NPB // NS)])
        plsc.subcore_barrier()
        return 0

    lax.fori_loop(0, NPASS, run_pass, 0)


@functools.partial(jax.jit, static_argnums=())
def _sc_edges(nodes, edges, tabs_flat, zrows):
    mesh = plsc.VectorSubcoreMesh(core_axis_name="c", subcore_axis_name="s")
    return pl.kernel(
        _sc_body,
        out_type=jax.ShapeDtypeStruct((NC, 4, NPAD, H), jnp.float32),
        mesh=mesh,
        compiler_params=pltpu.CompilerParams(needs_layout_passes=False),
        scratch_types=[
            pltpu.VMEM((BUF,), jnp.int32),
            pltpu.VMEM((BLK,), jnp.int32),
            pltpu.VMEM((BLK,), jnp.int32),
            pltpu.VMEM((K, REC), jnp.float32),
            pltpu.VMEM((K, REC), jnp.float32),
            pltpu.VMEM((K, H), jnp.float32),
            pltpu.VMEM((K, H), jnp.float32),
            pltpu.VMEM((K, H), jnp.float32),
            pltpu.VMEM((K, H), jnp.float32),
            pltpu.VMEM((K,), jnp.int32),
            pltpu.VMEM((K,), jnp.int32),
            pltpu.VMEM((K,), jnp.int32),
            pltpu.VMEM((5 * GR,), jnp.float32),
            pltpu.VMEM((2 * GR,), jnp.int32),
            pltpu.VMEM((4096,), jnp.float32),
            pltpu.SMEM((40,), jnp.int32),
            pltpu.VMEM_SHARED((ACC_ROWS, H), jnp.float32),
            pltpu.VMEM_SHARED((ACC_ROWS, H), jnp.float32),
            pltpu.VMEM_SHARED((ACC_ROWS, H), jnp.float32),
            pltpu.VMEM_SHARED((ACC_ROWS, H), jnp.float32),
            pltpu.SemaphoreType.DMA,
            pltpu.SemaphoreType.DMA,
        ],
    )(nodes, edges, tabs_flat, zrows)


def _post_body(partial_ref, species_ref, tabs_ref, watom_ref, wh_ref, bh_ref,
               wv3_ref, hout_ref, vflat_ref):
    f32 = jnp.float32
    p = partial_ref[...]
    g = p[0] + p[1]
    gm = g[0]
    gcat = jnp.concatenate([g[1], g[2], g[3]], axis=1)
    s = species_ref[0, 0, :]
    onehot = (s[:, None] == lax.broadcasted_iota(jnp.int32, (s.shape[0], 8), 1)
              ).astype(f32)
    h_atom = jnp.dot(onehot, watom_ref[...], preferred_element_type=f32)
    ht = tabs_ref[26:27, 0:EMBED]
    h = jnp.concatenate(
        [h_atom, jnp.broadcast_to(ht, (s.shape[0], EMBED))], axis=1)
    hout_ref[...] = (h + jnp.dot(gm, wh_ref[...],
                                 preferred_element_type=f32) + bh_ref[...])
    vflat_ref[...] = jnp.dot(gcat, wv3_ref[...], preferred_element_type=f32)


def _post(partial, species_r, tabs24, watom_pad, W_h, b_h, wv3):
    bn = 512
    nblk = NPAD // bn
    return pl.pallas_call(
        _post_body,
        grid=(nblk,),
        in_specs=[
            pl.BlockSpec((NC, 4, bn, H), lambda i: (0, 0, i, 0)),
            pl.BlockSpec((1, 1, bn), lambda i: (i, 0, 0)),
            pl.BlockSpec((32, 128), lambda i: (0, 0)),
            pl.BlockSpec((8, EMBED), lambda i: (0, 0)),
            pl.BlockSpec((H, H), lambda i: (0, 0)),
            pl.BlockSpec((1, H), lambda i: (0, 0)),
            pl.BlockSpec((3 * H, 3 * EMBED), lambda i: (0, 0)),
        ],
        out_specs=[
            pl.BlockSpec((bn, H), lambda i: (i, 0)),
            pl.BlockSpec((bn, 3 * EMBED), lambda i: (i, 0)),
        ],
        out_shape=[
            jax.ShapeDtypeStruct((NPAD, H), jnp.float32),
            jax.ShapeDtypeStruct((NPAD, 3 * EMBED), jnp.float32),
        ],
    )(partial, species_r, tabs24, watom_pad, W_h, b_h, wv3)


def kernel(species, x, edge_index, t, fragments_idx, W_atom, b_atom, B_rff,
           W1, b1, W2, b2, W_e, b_e, W_h, b_h, W_v):
    f32 = jnp.float32
    i32 = jnp.int32

    tabs24 = _prep(t, B_rff, W1, b1, W2, b2, W_atom, b_atom, W_e, b_e)
    tabs_flat = tabs24.reshape(4096)

    nodes = jnp.zeros((NODE_ROWS, REC), f32)
    nodes = nodes.at[:N, 0:3].set(x)
    nodes = nodes.at[:N, 3].set(species.astype(f32))
    nodes = nodes.at[:N, 4].set(fragments_idx.astype(f32))

    ei = edge_index.astype(i32)
    pad = jnp.concatenate(
        [jnp.zeros((1, E_PAD - E), i32),
         jnp.full((1, E_PAD - E), NPAD, i32)], axis=0)
    edges = jnp.concatenate([ei, pad], axis=1)

    zrows = jnp.zeros((ZROWS, H), f32)
    partial = _sc_edges(nodes, edges, tabs_flat, zrows)

    wv3 = jnp.zeros((3 * H, 3 * EMBED), f32)
    for c in range(3):
        wv3 = wv3.at[c * H:(c + 1) * H, c::3].set(W_v)

    species_r = jnp.zeros((NPAD,), i32).at[:N].set(species.astype(i32))
    species_r = species_r.reshape(NPAD // 512, 1, 512)
    watom_pad = jnp.concatenate([W_atom, jnp.zeros((3, EMBED), f32)], axis=0)

    h_out, v_flat = _post(partial, species_r, tabs24, watom_pad, W_h,
                          b_h.reshape(1, H), wv3)
    return (h_out[:N], v_flat[:N].reshape(N, EMBED, 3))
